# baseline (speedup 1.0000x reference)
import jax
import jax.numpy as jnp
from jax.experimental import pallas as pl
from jax.experimental.pallas import tpu as pltpu

_BN = 4096
_K = 512
_D = 64


def _assign_kernel(f_ref, c_ref, out_ref):
    f = f_ref[...]
    c = c_ref[...]
    norm = jnp.sqrt(jnp.sum(f * f, axis=1, keepdims=True))
    f = f / jnp.maximum(norm, 1e-12)
    sim = jax.lax.dot_general(
        f, c, (((1,), (1,)), ((), ())), preferred_element_type=jnp.float32
    )
    m = jnp.max(sim, axis=1, keepdims=True)
    ids = jax.lax.broadcasted_iota(jnp.int32, sim.shape, 1)
    idx = jnp.min(jnp.where(sim == m, ids, jnp.int32(_K)), axis=1)
    out_ref[...] = idx


def kernel(features, centroids):
    n = features.shape[0]
    grid = (n // _BN,)
    assignments = pl.pallas_call(
        _assign_kernel,
        grid=grid,
        in_specs=[
            pl.BlockSpec((_BN, _D), lambda i: (i, 0)),
            pl.BlockSpec((_K, _D), lambda i: (0, 0)),
        ],
        out_specs=pl.BlockSpec((_BN,), lambda i: (i,)),
        out_shape=jax.ShapeDtypeStruct((n,), jnp.int32),
        compiler_params=pltpu.CompilerParams(
            dimension_semantics=("arbitrary",),
        ),
    )(features, centroids)
    return assignments


# speedup vs baseline: 1.4498x; 1.4498x over previous
import jax
import jax.numpy as jnp
from jax.experimental import pallas as pl
from jax.experimental.pallas import tpu as pltpu

_BN = 4096
_K = 512
_D = 64


def _assign_kernel(f_ref, c_ref, out_ref):
    f = f_ref[...]
    c = c_ref[...]
    norm = jnp.sqrt(jnp.sum(f * f, axis=1, keepdims=True))
    f = f / jnp.maximum(norm, 1e-12)
    sim = jax.lax.dot_general(
        c, f, (((1,), (1,)), ((), ())), preferred_element_type=jnp.float32
    )
    m = jnp.max(sim, axis=0, keepdims=True)
    ids = jax.lax.broadcasted_iota(jnp.int32, (_K, 1), 0).astype(jnp.float32)
    idx = jnp.min(jnp.where(sim == m, ids, jnp.float32(_K)), axis=0)
    out_ref[...] = idx.astype(jnp.int32)


def kernel(features, centroids):
    n = features.shape[0]
    grid = (n // _BN,)
    assignments = pl.pallas_call(
        _assign_kernel,
        grid=grid,
        in_specs=[
            pl.BlockSpec((_BN, _D), lambda i: (i, 0)),
            pl.BlockSpec((_K, _D), lambda i: (0, 0)),
        ],
        out_specs=pl.BlockSpec((_BN,), lambda i: (i,)),
        out_shape=jax.ShapeDtypeStruct((n,), jnp.int32),
        compiler_params=pltpu.CompilerParams(
            dimension_semantics=("arbitrary",),
        ),
    )(features, centroids)
    return assignments


# speedup vs baseline: 2.0866x; 1.4392x over previous
import jax
import jax.numpy as jnp
from jax.experimental import pallas as pl
from jax.experimental.pallas import tpu as pltpu

_BN = 4096
_K = 512
_D = 64
_G = 8


def _assign_kernel(f_ref, c_ref, out_ref):
    f = f_ref[...]
    c = c_ref[...]
    sim = jax.lax.dot_general(
        c, f, (((1,), (1,)), ((), ())), preferred_element_type=jnp.float32
    )

    best_v = sim[0:_G, :]
    best_g = jnp.zeros((_G, _BN), jnp.float32)
    for g in range(1, _K // _G):
        v = sim[g * _G:(g + 1) * _G, :]
        take = v > best_v
        best_v = jnp.where(take, v, best_v)
        best_g = jnp.where(take, jnp.float32(g), best_g)

    m = jnp.max(best_v, axis=0, keepdims=True)
    r = jax.lax.broadcasted_iota(jnp.int32, (_G, 1), 0).astype(jnp.float32)
    idx = jnp.min(
        jnp.where(best_v == m, best_g * jnp.float32(_G) + r, jnp.float32(_K)),
        axis=0,
    )
    out_ref[...] = idx.astype(jnp.int32)


def kernel(features, centroids):
    n = features.shape[0]
    grid = (n // _BN,)
    assignments = pl.pallas_call(
        _assign_kernel,
        grid=grid,
        in_specs=[
            pl.BlockSpec((_BN, _D), lambda i: (i, 0)),
            pl.BlockSpec((_K, _D), lambda i: (0, 0)),
        ],
        out_specs=pl.BlockSpec((_BN,), lambda i: (i,)),
        out_shape=jax.ShapeDtypeStruct((n,), jnp.int32),
        compiler_params=pltpu.CompilerParams(
            dimension_semantics=("arbitrary",),
        ),
    )(features, centroids)
    return assignments
